# baseline (speedup 1.0000x reference)
import functools

import jax
import jax.numpy as jnp
from jax import lax
from jax.experimental import pallas as pl
from jax.experimental.pallas import tpu as pltpu
from jax.experimental.pallas import tpu_sc as plsc

EMB = 64
SEQ = 200
NC, NS, L = 2, 16, 16
NW = NC * NS
CHUNK = 128
PAIR = CHUNK // 2
NBUF = 5
POS2_ROWS = 160


def _emb_kernel(n_rows):
    rows_per_w = n_rows // NW
    n_chunks = rows_per_w // CHUNK
    n_groups = n_chunks // NBUF
    mesh = plsc.VectorSubcoreMesh(
        core_axis_name="c", subcore_axis_name="s", num_cores=NC, num_subcores=NS
    )

    @functools.partial(
        pl.kernel,
        out_type=jax.ShapeDtypeStruct((n_rows // 2, 128), jnp.float32),
        mesh=mesh,
        scratch_types=[
            pltpu.VMEM((n_chunks, CHUNK), jnp.int32),
            pltpu.VMEM((NBUF, CHUNK, 128), jnp.float32),
            pltpu.VMEM((POS2_ROWS, 128), jnp.float32),
            pltpu.SemaphoreType.DMA((NBUF,)),
            pltpu.SemaphoreType.DMA((NBUF,)),
        ],
        compiler_params=pltpu.CompilerParams(use_tc_tiling_on_sc=True),
    )
    def body(tok_hbm, tab_hbm, pos2_hbm, out_hbm, idx_all, gbuf, pos2_v,
             gsem, wsem):
        wid = lax.axis_index("s") * NC + lax.axis_index("c")
        pltpu.sync_copy(
            tok_hbm.at[pl.ds(pl.multiple_of(wid * n_chunks, 8), n_chunks)],
            idx_all)
        pltpu.sync_copy(pos2_hbm, pos2_v)

        def out_slice(i):
            off = (wid * rows_per_w + i * CHUNK) // 2
            return out_hbm.at[pl.ds(pl.multiple_of(off, 8), PAIR)]

        def gather(i, b):
            for k in range(CHUNK // L):
                pidx = idx_all[i, pl.ds(k * L, L)] >> 1
                pltpu.async_copy(
                    tab_hbm.at[pidx], gbuf.at[b, pl.ds(k * L, L)], gsem.at[b])

        def gather_wait(i, b):
            dummy = idx_all[i, pl.ds(0, L)]
            for k in range(CHUNK // L):
                pltpu.make_async_copy(
                    tab_hbm.at[dummy], gbuf.at[b, pl.ds(k * L, L)],
                    gsem.at[b]).wait()

        def write(i, b):
            pltpu.async_copy(gbuf.at[b, pl.ds(0, PAIR)], out_slice(i),
                             wsem.at[b])

        def write_wait(i, b):
            pltpu.make_async_copy(gbuf.at[b, pl.ds(0, PAIR)], out_slice(i),
                                  wsem.at[b]).wait()

        for b in range(NBUF - 1):
            gather(b, b)

        def group_body(q, carry):
            for b in range(NBUF):
                i = q * NBUF + b
                bp = (b + NBUF - 1) % NBUF

                @pl.when(i + NBUF - 1 < n_chunks)
                def _():
                    @pl.when(i >= 1)
                    def _():
                        write_wait(i - 1, bp)

                    gather(i + NBUF - 1, bp)

                gather_wait(i, b)
                off2 = ((i * CHUNK) % SEQ) // 2

                @pl.loop(0, PAIR // 8)
                def addgrp(r8):
                    hvec = (idx_all[i, pl.ds(r8 * 16, 16)] & 1) * EMB
                    for t in range(8):
                        r = r8 * 8 + t
                        h0 = hvec[2 * t]
                        h1 = hvec[2 * t + 1]
                        for j in range(EMB // L):
                            lo = gbuf[b, 2 * r, pl.ds(h0 + j * L, L)]
                            hi = gbuf[b, 2 * r + 1, pl.ds(h1 + j * L, L)]
                            gbuf[b, r, pl.ds(j * L, L)] = (
                                lo + pos2_v[off2 + r, pl.ds(j * L, L)])
                            gbuf[b, r, pl.ds(EMB + j * L, L)] = (
                                hi + pos2_v[off2 + r, pl.ds(EMB + j * L, L)])

                write(i, b)
            return carry

        lax.fori_loop(0, n_groups, group_body, 0)
        for b in range(NBUF):
            write_wait(n_chunks - NBUF + b, (n_chunks - NBUF + b) % NBUF)

    return body


def kernel(tokens, static_table, pos_table):
    b, s = tokens.shape
    toks = tokens.reshape(-1, CHUNK).astype(jnp.int32)
    tab2 = static_table.reshape(-1, 128)
    pos_pairs = pos_table.reshape(-1, 128)
    pos2 = jnp.concatenate([pos_pairs, pos_pairs[: POS2_ROWS - SEQ // 2]], 0)
    out = _emb_kernel(b * s)(toks, tab2, pos2)
    return out.reshape(b, s, EMB)
